# Initial kernel scaffold; baseline (speedup 1.0000x reference)
import jax
import jax.numpy as jnp
from jax.experimental import pallas as pl


def kernel(input_ids, embed_table, Vk):
    raise NotImplementedError("write your pallas kernel here")


# speedup vs baseline: 12.5147x; 12.5147x over previous
import functools

import jax
import jax.numpy as jnp
from jax import lax
from jax.experimental import pallas as pl
from jax.experimental.pallas import tpu as pltpu
from jax.experimental.pallas import tpu_sc as plsc

VOCAB = 1_000_000
D_MODEL = 32
K = 8
BATCH = 16384
HIST = 50

RG = 4
ROWS_TC = VOCAB // RG
BLOCK_TC = 10000

NC = 2
NS = 16
NW = NC * NS
IDX_COLS = 128
B_TOTAL = BATCH * HIST
IDX_ROWS = B_TOTAL // IDX_COLS
ROWS_PER_TILE = IDX_ROWS // NW
MACRO = 40
N_MACRO = ROWS_PER_TILE // MACRO


def _project_tc(table_r, vk_big):
    def body(x_ref, w_ref, o_ref):
        o_ref[...] = jnp.dot(x_ref[...], w_ref[...],
                             preferred_element_type=jnp.float32)

    return pl.pallas_call(
        body,
        grid=(ROWS_TC // BLOCK_TC,),
        in_specs=[
            pl.BlockSpec((BLOCK_TC, RG * D_MODEL), lambda i: (i, 0)),
            pl.BlockSpec((RG * D_MODEL, RG * K), lambda i: (0, 0)),
        ],
        out_specs=pl.BlockSpec((BLOCK_TC, RG * K), lambda i: (i, 0)),
        out_shape=jax.ShapeDtypeStruct((ROWS_TC, RG * K), jnp.float32),
    )(table_r, vk_big)


_MESH = plsc.VectorSubcoreMesh(core_axis_name="c", subcore_axis_name="s")


@functools.partial(
    pl.kernel,
    mesh=_MESH,
    compiler_params=pltpu.CompilerParams(use_tc_tiling_on_sc=False),
    out_type=jax.ShapeDtypeStruct((B_TOTAL, K), jnp.float32),
    scratch_types=[
        pltpu.VMEM((MACRO, IDX_COLS), jnp.int32),
        pltpu.VMEM((MACRO * IDX_COLS, K), jnp.float32),
        pltpu.SemaphoreType.DMA,
    ],
)
def _gather_sc(idx_hbm, p_hbm, out_hbm, idx_v, stage_v, sem):
    wid = lax.axis_index("s") * NC + lax.axis_index("c")
    for m in range(N_MACRO):
        row0 = wid * ROWS_PER_TILE + m * MACRO
        pltpu.sync_copy(idx_hbm.at[pl.ds(row0, MACRO)], idx_v)

        def fire(j, carry):
            pltpu.make_async_copy(
                p_hbm.at[idx_v.at[j]],
                stage_v.at[pl.ds(j * IDX_COLS, IDX_COLS)],
                sem,
            ).start()
            return carry

        lax.fori_loop(0, MACRO, fire, 0)
        pltpu.make_async_copy(
            out_hbm.at[pl.ds(0, MACRO * IDX_COLS)], stage_v, sem
        ).wait()
        pltpu.sync_copy(
            stage_v, out_hbm.at[pl.ds(row0 * IDX_COLS, MACRO * IDX_COLS)]
        )


def kernel(input_ids, embed_table, Vk):
    table_r = embed_table.reshape(ROWS_TC, RG * D_MODEL)
    vk_big = jnp.kron(jnp.eye(RG, dtype=Vk.dtype), Vk)
    p = _project_tc(table_r, vk_big).reshape(VOCAB, K)
    idx = input_ids.reshape(IDX_ROWS, IDX_COLS)
    out = _gather_sc(idx, p)
    return out.reshape(BATCH, HIST, K)


# speedup vs baseline: 22.9854x; 1.8367x over previous
import functools

import jax
import jax.numpy as jnp
from jax import lax
from jax.experimental import pallas as pl
from jax.experimental.pallas import tpu as pltpu
from jax.experimental.pallas import tpu_sc as plsc

VOCAB = 1_000_000
D_MODEL = 32
K = 8
BATCH = 16384
HIST = 50

RG = 4
ROWS_TC = VOCAB // RG
BLOCK_TC = 10000

NC = 2
NS = 16
NW = NC * NS
IDX_COLS = 128
B_TOTAL = BATCH * HIST
IDX_ROWS = B_TOTAL // IDX_COLS
ROWS_PER_TILE = IDX_ROWS // NW
MACRO = 40
N_MACRO = ROWS_PER_TILE // MACRO


def _project_tc(table_r, vk_big):
    def body(x_ref, w_ref, o_ref):
        o_ref[...] = jnp.dot(x_ref[...], w_ref[...],
                             preferred_element_type=jnp.float32)

    return pl.pallas_call(
        body,
        grid=(ROWS_TC // BLOCK_TC,),
        in_specs=[
            pl.BlockSpec((BLOCK_TC, RG * D_MODEL), lambda i: (i, 0)),
            pl.BlockSpec((RG * D_MODEL, RG * K), lambda i: (0, 0)),
        ],
        out_specs=pl.BlockSpec((BLOCK_TC, RG * K), lambda i: (i, 0)),
        out_shape=jax.ShapeDtypeStruct((ROWS_TC, RG * K), jnp.float32),
    )(table_r, vk_big)


_MESH = plsc.VectorSubcoreMesh(core_axis_name="c", subcore_axis_name="s")

M_ROWS = MACRO * IDX_COLS
M_GROUPS = M_ROWS // 16


@functools.partial(
    pl.kernel,
    mesh=_MESH,
    compiler_params=pltpu.CompilerParams(
        use_tc_tiling_on_sc=False, needs_layout_passes=False
    ),
    out_type=jax.ShapeDtypeStruct((K, B_TOTAL), jnp.float32),
    scratch_types=[
        pltpu.VMEM((MACRO, IDX_COLS), jnp.int32),
        pltpu.VMEM((M_ROWS, K), jnp.float32),
        pltpu.VMEM((K, M_ROWS), jnp.float32),
        pltpu.SemaphoreType.DMA,
    ],
)
def _gather_sc(idx_hbm, p_hbm, out_hbm, idx_v, stage_v, staget_v, sem):
    wid = lax.axis_index("s") * NC + lax.axis_index("c")
    lanes = lax.iota(jnp.int32, 16)
    for m in range(N_MACRO):
        row0 = wid * ROWS_PER_TILE + m * MACRO
        pltpu.sync_copy(idx_hbm.at[pl.ds(row0, MACRO)], idx_v)

        def fire(j, carry):
            pltpu.make_async_copy(
                p_hbm.at[idx_v.at[j]],
                stage_v.at[pl.ds(j * IDX_COLS, IDX_COLS)],
                sem,
            ).start()
            return carry

        lax.fori_loop(0, MACRO, fire, 0)
        pltpu.make_async_copy(
            p_hbm.at[pl.ds(0, M_ROWS)], stage_v, sem
        ).wait()

        def transp(g, carry):
            rows = g * 16 + lanes
            for k in range(K):
                cols = jnp.full((16,), k, jnp.int32)
                v = plsc.load_gather(stage_v, [rows, cols])
                staget_v[k, pl.ds(g * 16, 16)] = v
            return carry

        lax.fori_loop(0, M_GROUPS, transp, 0)
        for k in range(K):
            pltpu.sync_copy(
                staget_v.at[k],
                out_hbm.at[k, pl.ds(row0 * IDX_COLS, M_ROWS)],
            )


def kernel(input_ids, embed_table, Vk):
    table_r = embed_table.reshape(ROWS_TC, RG * D_MODEL)
    vk_big = jnp.kron(jnp.eye(RG, dtype=Vk.dtype), Vk)
    p = _project_tc(table_r, vk_big).reshape(VOCAB, K)
    idx = input_ids.reshape(IDX_ROWS, IDX_COLS)
    out_t = _gather_sc(idx, p)
    out_t = out_t.reshape(K, BATCH, HIST)
    return jnp.transpose(out_t, (1, 2, 0))


# speedup vs baseline: 25.9686x; 1.1298x over previous
import functools

import jax
import jax.numpy as jnp
from jax import lax
from jax.experimental import pallas as pl
from jax.experimental.pallas import tpu as pltpu
from jax.experimental.pallas import tpu_sc as plsc

VOCAB = 1_000_000
D_MODEL = 32
K = 8
BATCH = 16384
HIST = 50

NC = 2
NS = 16
NW = NC * NS
IDX_COLS = 128
B_TOTAL = BATCH * HIST
IDX_ROWS = B_TOTAL // IDX_COLS
ROWS_PER_TILE = IDX_ROWS // NW
MACRO = 40
N_MACRO = ROWS_PER_TILE // MACRO


BN = 16384


def _project_tc(table_t, vk):
    def body(x_ref, w_ref, o_ref):
        o_ref[...] = jax.lax.dot_general(
            x_ref[...], w_ref[...],
            dimension_numbers=(((0,), (0,)), ((), ())),
            preferred_element_type=jnp.float32,
        )

    return pl.pallas_call(
        body,
        grid=(pl.cdiv(VOCAB, BN),),
        in_specs=[
            pl.BlockSpec((D_MODEL, BN), lambda i: (0, i)),
            pl.BlockSpec((D_MODEL, K), lambda i: (0, 0)),
        ],
        out_specs=pl.BlockSpec((BN, K), lambda i: (i, 0)),
        out_shape=jax.ShapeDtypeStruct((VOCAB, K), jnp.float32),
    )(table_t, vk)


_MESH = plsc.VectorSubcoreMesh(core_axis_name="c", subcore_axis_name="s")

M_ROWS = MACRO * IDX_COLS
M_GROUPS = M_ROWS // 16


@functools.partial(
    pl.kernel,
    mesh=_MESH,
    compiler_params=pltpu.CompilerParams(
        use_tc_tiling_on_sc=False, needs_layout_passes=False
    ),
    out_type=jax.ShapeDtypeStruct((K, B_TOTAL), jnp.float32),
    scratch_types=[
        pltpu.VMEM((MACRO, IDX_COLS), jnp.int32),
        pltpu.VMEM((M_ROWS, K), jnp.float32),
        pltpu.VMEM((K, M_ROWS), jnp.float32),
        pltpu.SemaphoreType.DMA,
    ],
)
def _gather_sc(idx_hbm, p_hbm, out_hbm, idx_v, stage_v, staget_v, sem):
    wid = lax.axis_index("s") * NC + lax.axis_index("c")
    lanes = lax.iota(jnp.int32, 16)
    for m in range(N_MACRO):
        row0 = wid * ROWS_PER_TILE + m * MACRO
        pltpu.sync_copy(idx_hbm.at[pl.ds(row0, MACRO)], idx_v)

        def fire(j, carry):
            pltpu.make_async_copy(
                p_hbm.at[idx_v.at[j]],
                stage_v.at[pl.ds(j * IDX_COLS, IDX_COLS)],
                sem,
            ).start()
            return carry

        lax.fori_loop(0, MACRO, fire, 0)
        pltpu.make_async_copy(
            p_hbm.at[pl.ds(0, M_ROWS)], stage_v, sem
        ).wait()

        def transp(g, carry):
            rows = g * 16 + lanes
            for k in range(K):
                cols = jnp.full((16,), k, jnp.int32)
                v = plsc.load_gather(stage_v, [rows, cols])
                staget_v[k, pl.ds(g * 16, 16)] = v
            return carry

        lax.fori_loop(0, M_GROUPS, transp, 0)
        for k in range(K):
            pltpu.sync_copy(
                staget_v.at[k],
                out_hbm.at[k, pl.ds(row0 * IDX_COLS, M_ROWS)],
            )


def kernel(input_ids, embed_table, Vk):
    table_t = embed_table.T
    p = _project_tc(table_t, Vk)
    idx = input_ids.reshape(IDX_ROWS, IDX_COLS)
    out_t = _gather_sc(idx, p)
    out_t = out_t.reshape(K, BATCH, HIST)
    return jnp.transpose(out_t, (1, 2, 0))
